# Initial kernel scaffold; baseline (speedup 1.0000x reference)
import jax
import jax.numpy as jnp
from jax.experimental import pallas as pl


def kernel(x, edge_index, edge_attr, batch, W_emb, b_emb, bond_W, bond_b, mlp_W1, mlp_b1, mlp_g, mlp_beta, mlp_W2, mlp_b2, bn_g, bn_beta):
    raise NotImplementedError("write your pallas kernel here")


# speedup vs baseline: 2.6485x; 2.6485x over previous
import functools

import jax
import jax.numpy as jnp
from jax import lax
from jax.experimental import pallas as pl
from jax.experimental.pallas import tpu as pltpu
from jax.experimental.pallas import tpu_sc as plsc

N = 10000
E = 640000
ATOM = 101
BOND = 11
D = 128
L = 3
G = 256

NC = 2
NS = 16
NW = NC * NS
CH = 128
EPAD = NW * CH * 157
TPT = EPAD // NW
NCHUNK = TPT // CH
NP = 10240
RPT = NP // NS
F32 = jnp.float32


def _embed(x, w, b):
    def body(x_ref, w_ref, b_ref, o_ref):
        o_ref[:] = (
            jnp.dot(x_ref[:], w_ref[:], preferred_element_type=F32) + b_ref[:]
        )

    R = 1000
    return pl.pallas_call(
        body,
        grid=(N // R,),
        in_specs=[
            pl.BlockSpec((R, ATOM), lambda i: (i, 0)),
            pl.BlockSpec((ATOM, D), lambda i: (0, 0)),
            pl.BlockSpec((1, D), lambda i: (0, 0)),
        ],
        out_specs=pl.BlockSpec((R, D), lambda i: (i, 0)),
        out_shape=jax.ShapeDtypeStruct((N, D), F32),
    )(x, w, b)


def _bond(attr, bond_W, bond_b):
    B = 4096

    def body(a_ref, w_ref, b_ref, o_ref):
        o_ref[0] = (
            jnp.dot(a_ref[:], w_ref[0], preferred_element_type=F32) + b_ref[0]
        )

    return pl.pallas_call(
        body,
        grid=(L, EPAD // B),
        in_specs=[
            pl.BlockSpec((B, BOND), lambda l, j: (j, 0)),
            pl.BlockSpec((1, BOND, D), lambda l, j: (l, 0, 0)),
            pl.BlockSpec((1, 1, D), lambda l, j: (l, 0, 0)),
        ],
        out_specs=pl.BlockSpec((1, B, D), lambda l, j: (l, j, 0)),
        out_shape=jax.ShapeDtypeStruct((L, EPAD, D), F32),
    )(attr, bond_W, bond_b)


R = 1000
NB = N // R


def _bn_stats(vals, i, bm_ref, m2_ref):
    mb = jnp.mean(vals, axis=0, keepdims=True)
    c = vals - mb
    bm_ref[0] = mb
    m2 = jnp.sum(c * c, axis=0, keepdims=True)

    @pl.when(i == 0)
    def _():
        m2_ref[:] = jnp.zeros_like(m2_ref)

    m2_ref[:] += m2


def _bn_apply(vals, bm, m2, g, beta):
    mean = jnp.mean(bm, axis=0, keepdims=True)
    dm = bm - mean
    var = m2 * (1.0 / N) + jnp.mean(dm * dm, axis=0, keepdims=True)
    return (vals - mean) * lax.rsqrt(var + 1e-5) * g + beta


def _mlp1(h, aggr, w1, b1):
    def body(h_ref, a_ref, w_ref, b_ref, z_ref, bm_ref, m2_ref):
        z = h_ref[:] + a_ref[0] + a_ref[1]
        z1 = jnp.dot(z, w_ref[:], preferred_element_type=F32) + b_ref[:]
        z_ref[:] = z1
        _bn_stats(z1, pl.program_id(0), bm_ref, m2_ref)

    return pl.pallas_call(
        body,
        grid=(NB,),
        in_specs=[
            pl.BlockSpec((R, D), lambda i: (i, 0)),
            pl.BlockSpec((2, R, D), lambda i: (0, i, 0)),
            pl.BlockSpec((D, 2 * D), lambda i: (0, 0)),
            pl.BlockSpec((1, 2 * D), lambda i: (0, 0)),
        ],
        out_specs=[
            pl.BlockSpec((R, 2 * D), lambda i: (i, 0)),
            pl.BlockSpec((1, 1, 2 * D), lambda i: (i, 0, 0)),
            pl.BlockSpec((1, 2 * D), lambda i: (0, 0)),
        ],
        out_shape=[
            jax.ShapeDtypeStruct((N, 2 * D), F32),
            jax.ShapeDtypeStruct((NB, 1, 2 * D), F32),
            jax.ShapeDtypeStruct((1, 2 * D), F32),
        ],
    )(h, aggr, w1, b1)


def _mlp2(z1, bm1, m21, g, beta, w2, b2):
    def body(z_ref, bm1_ref, m21_ref, g_ref, be_ref, w_ref, b_ref,
             o_ref, bm_ref, m2_ref):
        zn = _bn_apply(z_ref[:], bm1_ref[:], m21_ref[:], g_ref[:], be_ref[:])
        r = jnp.maximum(zn, 0.0)
        z2 = jnp.dot(r, w_ref[:], preferred_element_type=F32) + b_ref[:]
        o_ref[:] = z2
        _bn_stats(z2, pl.program_id(0), bm_ref, m2_ref)

    return pl.pallas_call(
        body,
        grid=(NB,),
        in_specs=[
            pl.BlockSpec((R, 2 * D), lambda i: (i, 0)),
            pl.BlockSpec((NB, 2 * D), lambda i: (0, 0)),
            pl.BlockSpec((1, 2 * D), lambda i: (0, 0)),
            pl.BlockSpec((1, 2 * D), lambda i: (0, 0)),
            pl.BlockSpec((1, 2 * D), lambda i: (0, 0)),
            pl.BlockSpec((2 * D, D), lambda i: (0, 0)),
            pl.BlockSpec((1, D), lambda i: (0, 0)),
        ],
        out_specs=[
            pl.BlockSpec((R, D), lambda i: (i, 0)),
            pl.BlockSpec((1, 1, D), lambda i: (i, 0, 0)),
            pl.BlockSpec((1, D), lambda i: (0, 0)),
        ],
        out_shape=[
            jax.ShapeDtypeStruct((N, D), F32),
            jax.ShapeDtypeStruct((NB, 1, D), F32),
            jax.ShapeDtypeStruct((1, D), F32),
        ],
    )(z1, bm1, m21, g, beta, w2, b2)


def _mlp3(z2, bm2, m22, g, beta, relu):
    def body(z_ref, bm_ref, m2_ref, g_ref, be_ref, o_ref):
        hn = _bn_apply(z_ref[:], bm_ref[:], m2_ref[:], g_ref[:], be_ref[:])
        if relu:
            hn = jnp.maximum(hn, 0.0)
        o_ref[:] = hn

    return pl.pallas_call(
        body,
        grid=(NB,),
        in_specs=[
            pl.BlockSpec((R, D), lambda i: (i, 0)),
            pl.BlockSpec((NB, D), lambda i: (0, 0)),
            pl.BlockSpec((1, D), lambda i: (0, 0)),
            pl.BlockSpec((1, D), lambda i: (0, 0)),
            pl.BlockSpec((1, D), lambda i: (0, 0)),
        ],
        out_specs=pl.BlockSpec((R, D), lambda i: (i, 0)),
        out_shape=jax.ShapeDtypeStruct((N, D), F32),
    )(z2, bm2, m22, g, beta)


def _pool(h, batch2d):
    GB = 64

    def body(h_ref, b_ref, o_ref):
        gbase = pl.program_id(0) * GB
        rows = lax.broadcasted_iota(jnp.int32, (GB, N), 0) + gbase
        mask = (rows == b_ref[:]).astype(F32)
        sums = jnp.dot(mask, h_ref[:], preferred_element_type=F32)
        counts = jnp.sum(mask, axis=1, keepdims=True)
        o_ref[:] = sums / jnp.maximum(counts, 1.0)

    return pl.pallas_call(
        body,
        grid=(G // GB,),
        in_specs=[
            pl.BlockSpec((N, D), lambda i: (0, 0)),
            pl.BlockSpec((1, N), lambda i: (0, 0)),
        ],
        out_specs=pl.BlockSpec((GB, D), lambda i: (i, 0)),
        out_shape=jax.ShapeDtypeStruct((G, D), F32),
    )(h, batch2d)


def _messages(h, e, src, dst):
    mesh = plsc.VectorSubcoreMesh(core_axis_name="c", subcore_axis_name="s")

    @functools.partial(
        pl.kernel,
        mesh=mesh,
        out_type=jax.ShapeDtypeStruct((NC, NP, D), F32),
        scratch_types=[
            pltpu.VMEM((CH,), jnp.int32),
            pltpu.VMEM((CH,), jnp.int32),
            pltpu.VMEM((CH, D), F32),
            pltpu.VMEM((CH, D), F32),
            pltpu.VMEM_SHARED((NP, D), F32),
            pltpu.SemaphoreType.DMA,
        ],
    )
    def k(h_hbm, e_hbm, src_hbm, dst_hbm, out_hbm,
          src_v, dst_v, hrows, erows, aggr_sh, sem):
        cid = lax.axis_index("c")
        sid = lax.axis_index("s")
        wid = cid * NS + sid

        def zb(r, carry):
            for j in range(D // 16):
                erows[r, pl.ds(j * 16, 16)] = jnp.zeros((16,), F32)
            return carry

        lax.fori_loop(0, CH, zb, 0)
        for t in range(RPT // CH):
            pltpu.sync_copy(erows, aggr_sh.at[pl.ds(sid * RPT + t * CH, CH)])
        plsc.subcore_barrier()

        def chunk(c, carry):
            off = wid * TPT + c * CH
            pltpu.sync_copy(src_hbm.at[pl.ds(off, CH)], src_v)
            pltpu.sync_copy(dst_hbm.at[pl.ds(off, CH)], dst_v)
            pltpu.async_copy(h_hbm.at[src_v], hrows, sem).wait()
            pltpu.sync_copy(e_hbm.at[pl.ds(off, CH)], erows)

            def msg(r, c2):
                for j in range(D // 16):
                    s = pl.ds(j * 16, 16)
                    erows[r, s] = jnp.maximum(hrows[r, s] + erows[r, s], 0.0)
                return c2

            lax.fori_loop(0, CH, msg, 0)
            pltpu.sync_copy(erows, aggr_sh.at[dst_v], add=True)
            return carry

        lax.fori_loop(0, NCHUNK, chunk, 0)
        plsc.subcore_barrier()
        for t in range(RPT // CH):
            r0 = sid * RPT + t * CH
            pltpu.sync_copy(aggr_sh.at[pl.ds(r0, CH)],
                            out_hbm.at[cid, pl.ds(r0, CH)])

    return k(h, e, src, dst)


def kernel(x, edge_index, edge_attr, batch, W_emb, b_emb, bond_W, bond_b,
           mlp_W1, mlp_b1, mlp_g, mlp_beta, mlp_W2, mlp_b2, bn_g, bn_beta):
    pad = EPAD - E
    src = jnp.concatenate([edge_index[0], jnp.zeros((pad,), jnp.int32)])
    dst = jnp.concatenate([edge_index[1], jnp.full((pad,), N, jnp.int32)])
    attr = jnp.concatenate(
        [edge_attr, jnp.zeros((pad, BOND), F32)], axis=0)

    h = _embed(x, W_emb, b_emb.reshape(1, D))
    e_all = _bond(attr, bond_W, bond_b.reshape(L, 1, D))

    for i in range(L):
        aggr = _messages(h, e_all[i], src, dst)
        z1, bm1, m21 = _mlp1(h, aggr, mlp_W1[i], mlp_b1[i].reshape(1, 2 * D))
        z2, bm2, m22 = _mlp2(z1, bm1.reshape(NB, 2 * D), m21,
                             mlp_g[i].reshape(1, 2 * D),
                             mlp_beta[i].reshape(1, 2 * D), mlp_W2[i],
                             mlp_b2[i].reshape(1, D))
        h = _mlp3(z2, bm2.reshape(NB, D), m22, bn_g[i].reshape(1, D),
                  bn_beta[i].reshape(1, D), relu=(i < L - 1))

    return _pool(h, batch.reshape(1, N))


# speedup vs baseline: 4.3813x; 1.6543x over previous
import functools

import jax
import jax.numpy as jnp
from jax import lax
from jax.experimental import pallas as pl
from jax.experimental.pallas import tpu as pltpu
from jax.experimental.pallas import tpu_sc as plsc

N = 10000
E = 640000
ATOM = 101
BOND = 11
D = 128
L = 3
G = 256

NC = 2
NS = 16
NW = NC * NS
CH = 64
EPAD = 643072
TPT = EPAD // NW
NCHUNK = TPT // CH
NPAIR = NCHUNK // 2
NP = 10240
RPT = NP // NS
F32 = jnp.float32


def _embed(x, w, b):
    def body(x_ref, w_ref, b_ref, o_ref):
        o_ref[:] = (
            jnp.dot(x_ref[:], w_ref[:], preferred_element_type=F32) + b_ref[:]
        )

    R = 1000
    return pl.pallas_call(
        body,
        grid=(N // R,),
        in_specs=[
            pl.BlockSpec((R, ATOM), lambda i: (i, 0)),
            pl.BlockSpec((ATOM, D), lambda i: (0, 0)),
            pl.BlockSpec((1, D), lambda i: (0, 0)),
        ],
        out_specs=pl.BlockSpec((R, D), lambda i: (i, 0)),
        out_shape=jax.ShapeDtypeStruct((N, D), F32),
    )(x, w, b)


def _bond(attr, bond_W, bond_b):
    B = 4096

    def body(a_ref, w_ref, b_ref, o_ref):
        o_ref[0] = (
            jnp.dot(a_ref[:], w_ref[0], preferred_element_type=F32) + b_ref[0]
        )

    return pl.pallas_call(
        body,
        grid=(L, EPAD // B),
        in_specs=[
            pl.BlockSpec((B, BOND), lambda l, j: (j, 0)),
            pl.BlockSpec((1, BOND, D), lambda l, j: (l, 0, 0)),
            pl.BlockSpec((1, 1, D), lambda l, j: (l, 0, 0)),
        ],
        out_specs=pl.BlockSpec((1, B, D), lambda l, j: (l, j, 0)),
        out_shape=jax.ShapeDtypeStruct((L, EPAD, D), F32),
    )(attr, bond_W, bond_b)


R = 1000
NB = N // R


def _bn_stats(vals, i, bm_ref, m2_ref):
    mb = jnp.mean(vals, axis=0, keepdims=True)
    c = vals - mb
    bm_ref[0] = mb
    m2 = jnp.sum(c * c, axis=0, keepdims=True)

    @pl.when(i == 0)
    def _():
        m2_ref[:] = jnp.zeros_like(m2_ref)

    m2_ref[:] += m2


def _bn_apply(vals, bm, m2, g, beta):
    mean = jnp.mean(bm, axis=0, keepdims=True)
    dm = bm - mean
    var = m2 * (1.0 / N) + jnp.mean(dm * dm, axis=0, keepdims=True)
    return (vals - mean) * lax.rsqrt(var + 1e-5) * g + beta


def _mlp1(h, aggr, w1, b1):
    def body(h_ref, a_ref, w_ref, b_ref, z_ref, bm_ref, m2_ref):
        z = h_ref[:] + a_ref[0] + a_ref[1]
        z1 = jnp.dot(z, w_ref[:], preferred_element_type=F32) + b_ref[:]
        z_ref[:] = z1
        _bn_stats(z1, pl.program_id(0), bm_ref, m2_ref)

    return pl.pallas_call(
        body,
        grid=(NB,),
        in_specs=[
            pl.BlockSpec((R, D), lambda i: (i, 0)),
            pl.BlockSpec((2, R, D), lambda i: (0, i, 0)),
            pl.BlockSpec((D, 2 * D), lambda i: (0, 0)),
            pl.BlockSpec((1, 2 * D), lambda i: (0, 0)),
        ],
        out_specs=[
            pl.BlockSpec((R, 2 * D), lambda i: (i, 0)),
            pl.BlockSpec((1, 1, 2 * D), lambda i: (i, 0, 0)),
            pl.BlockSpec((1, 2 * D), lambda i: (0, 0)),
        ],
        out_shape=[
            jax.ShapeDtypeStruct((N, 2 * D), F32),
            jax.ShapeDtypeStruct((NB, 1, 2 * D), F32),
            jax.ShapeDtypeStruct((1, 2 * D), F32),
        ],
    )(h, aggr, w1, b1)


def _mlp2(z1, bm1, m21, g, beta, w2, b2):
    def body(z_ref, bm1_ref, m21_ref, g_ref, be_ref, w_ref, b_ref,
             o_ref, bm_ref, m2_ref):
        zn = _bn_apply(z_ref[:], bm1_ref[:], m21_ref[:], g_ref[:], be_ref[:])
        r = jnp.maximum(zn, 0.0)
        z2 = jnp.dot(r, w_ref[:], preferred_element_type=F32) + b_ref[:]
        o_ref[:] = z2
        _bn_stats(z2, pl.program_id(0), bm_ref, m2_ref)

    return pl.pallas_call(
        body,
        grid=(NB,),
        in_specs=[
            pl.BlockSpec((R, 2 * D), lambda i: (i, 0)),
            pl.BlockSpec((NB, 2 * D), lambda i: (0, 0)),
            pl.BlockSpec((1, 2 * D), lambda i: (0, 0)),
            pl.BlockSpec((1, 2 * D), lambda i: (0, 0)),
            pl.BlockSpec((1, 2 * D), lambda i: (0, 0)),
            pl.BlockSpec((2 * D, D), lambda i: (0, 0)),
            pl.BlockSpec((1, D), lambda i: (0, 0)),
        ],
        out_specs=[
            pl.BlockSpec((R, D), lambda i: (i, 0)),
            pl.BlockSpec((1, 1, D), lambda i: (i, 0, 0)),
            pl.BlockSpec((1, D), lambda i: (0, 0)),
        ],
        out_shape=[
            jax.ShapeDtypeStruct((N, D), F32),
            jax.ShapeDtypeStruct((NB, 1, D), F32),
            jax.ShapeDtypeStruct((1, D), F32),
        ],
    )(z1, bm1, m21, g, beta, w2, b2)


def _mlp3(z2, bm2, m22, g, beta, relu):
    def body(z_ref, bm_ref, m2_ref, g_ref, be_ref, o_ref):
        hn = _bn_apply(z_ref[:], bm_ref[:], m2_ref[:], g_ref[:], be_ref[:])
        if relu:
            hn = jnp.maximum(hn, 0.0)
        o_ref[:] = hn

    return pl.pallas_call(
        body,
        grid=(NB,),
        in_specs=[
            pl.BlockSpec((R, D), lambda i: (i, 0)),
            pl.BlockSpec((NB, D), lambda i: (0, 0)),
            pl.BlockSpec((1, D), lambda i: (0, 0)),
            pl.BlockSpec((1, D), lambda i: (0, 0)),
            pl.BlockSpec((1, D), lambda i: (0, 0)),
        ],
        out_specs=pl.BlockSpec((R, D), lambda i: (i, 0)),
        out_shape=jax.ShapeDtypeStruct((N, D), F32),
    )(z2, bm2, m22, g, beta)


def _pool(h, batch2d):
    GB = 64

    def body(h_ref, b_ref, o_ref):
        gbase = pl.program_id(0) * GB
        rows = lax.broadcasted_iota(jnp.int32, (GB, N), 0) + gbase
        mask = (rows == b_ref[:]).astype(F32)
        sums = jnp.dot(mask, h_ref[:], preferred_element_type=F32)
        counts = jnp.sum(mask, axis=1, keepdims=True)
        o_ref[:] = sums / jnp.maximum(counts, 1.0)

    return pl.pallas_call(
        body,
        grid=(G // GB,),
        in_specs=[
            pl.BlockSpec((N, D), lambda i: (0, 0)),
            pl.BlockSpec((1, N), lambda i: (0, 0)),
        ],
        out_specs=pl.BlockSpec((GB, D), lambda i: (i, 0)),
        out_shape=jax.ShapeDtypeStruct((G, D), F32),
    )(h, batch2d)


def _messages(h, e_all, src, dst, layer):
    mesh = plsc.VectorSubcoreMesh(core_axis_name="c", subcore_axis_name="s")

    @functools.partial(
        pl.kernel,
        mesh=mesh,
        out_type=jax.ShapeDtypeStruct((NC, NP, D), F32),
        scratch_types=[
            pltpu.VMEM((CH,), jnp.int32),
            pltpu.VMEM((CH,), jnp.int32),
            pltpu.VMEM((CH,), jnp.int32),
            pltpu.VMEM((CH,), jnp.int32),
            pltpu.VMEM((CH, D), F32),
            pltpu.VMEM((CH, D), F32),
            pltpu.VMEM((CH, D), F32),
            pltpu.VMEM((CH, D), F32),
            pltpu.VMEM_SHARED((NP, D), F32),
        ] + [pltpu.SemaphoreType.DMA] * 8,
    )
    def k(h_hbm, e_hbm, src_hbm, dst_hbm, out_hbm,
          sv0, dv0, sv1, dv1, h0, e0, h1, e1, aggr_sh,
          is0, id0, ig0, ie0, is1, id1, ig1, ie1):
        cid = lax.axis_index("c")
        sid = lax.axis_index("s")
        base = (cid * NS + sid) * TPT

        def zb(r, carry):
            for j in range(D // 16):
                e0[r, pl.ds(j * 16, 16)] = jnp.zeros((16,), F32)
            return carry

        lax.fori_loop(0, CH, zb, 0)
        for t in range(RPT // CH):
            pltpu.sync_copy(e0, aggr_sh.at[pl.ds(sid * RPT + t * CH, CH)])
        plsc.subcore_barrier()

        def fire(c, sv, dv, hb, eb, s_is, s_id, s_ig, s_ie, gather_only):
            off = base + c * CH
            if not gather_only:
                pltpu.make_async_copy(
                    src_hbm.at[pl.ds(off, CH)], sv, s_is).start()
                pltpu.make_async_copy(
                    dst_hbm.at[pl.ds(off, CH)], dv, s_id).start()
                pltpu.make_async_copy(
                    e_hbm.at[layer, pl.ds(off, CH)], eb, s_ie).start()
            else:
                pltpu.make_async_copy(
                    src_hbm.at[pl.ds(off, CH)], sv, s_is).wait()
                pltpu.make_async_copy(h_hbm.at[sv], hb, s_ig).start()

        def process(c, sv, dv, hb, eb, s_is, s_id, s_ig, s_ie):
            off = base + c * CH
            pltpu.make_async_copy(h_hbm.at[sv], hb, s_ig).wait()
            pltpu.make_async_copy(
                e_hbm.at[layer, pl.ds(off, CH)], eb, s_ie).wait()

            def msg(r, c2):
                for j in range(D // 16):
                    s = pl.ds(j * 16, 16)
                    eb[r, s] = jnp.maximum(hb[r, s] + eb[r, s], 0.0)
                return c2

            lax.fori_loop(0, CH, msg, 0)
            pltpu.make_async_copy(
                dst_hbm.at[pl.ds(off, CH)], dv, s_id).wait()
            pltpu.sync_copy(eb, aggr_sh.at[dv], add=True)

        fire(0, sv0, dv0, h0, e0, is0, id0, ig0, ie0, False)
        fire(1, sv1, dv1, h1, e1, is1, id1, ig1, ie1, False)
        fire(0, sv0, dv0, h0, e0, is0, id0, ig0, ie0, True)
        fire(1, sv1, dv1, h1, e1, is1, id1, ig1, ie1, True)

        def pair(p, carry):
            a = 2 * p
            process(a, sv0, dv0, h0, e0, is0, id0, ig0, ie0)

            @pl.when(p < NPAIR - 1)
            def _():
                fire(a + 2, sv0, dv0, h0, e0, is0, id0, ig0, ie0, False)
                fire(a + 2, sv0, dv0, h0, e0, is0, id0, ig0, ie0, True)

            process(a + 1, sv1, dv1, h1, e1, is1, id1, ig1, ie1)

            @pl.when(p < NPAIR - 1)
            def _():
                fire(a + 3, sv1, dv1, h1, e1, is1, id1, ig1, ie1, False)
                fire(a + 3, sv1, dv1, h1, e1, is1, id1, ig1, ie1, True)

            return carry

        lax.fori_loop(0, NPAIR, pair, 0)
        plsc.subcore_barrier()
        for t in range(RPT // CH):
            r0 = sid * RPT + t * CH
            pltpu.sync_copy(aggr_sh.at[pl.ds(r0, CH)],
                            out_hbm.at[cid, pl.ds(r0, CH)])

    return k(h, e_all, src, dst)


def kernel(x, edge_index, edge_attr, batch, W_emb, b_emb, bond_W, bond_b,
           mlp_W1, mlp_b1, mlp_g, mlp_beta, mlp_W2, mlp_b2, bn_g, bn_beta):
    pad = EPAD - E
    src = jnp.concatenate([edge_index[0], jnp.zeros((pad,), jnp.int32)])
    dst = jnp.concatenate([edge_index[1], jnp.full((pad,), N, jnp.int32)])
    attr = jnp.concatenate(
        [edge_attr, jnp.zeros((pad, BOND), F32)], axis=0)

    h = _embed(x, W_emb, b_emb.reshape(1, D))
    e_all = _bond(attr, bond_W, bond_b.reshape(L, 1, D))

    for i in range(L):
        aggr = _messages(h, e_all, src, dst, i)
        z1, bm1, m21 = _mlp1(h, aggr, mlp_W1[i], mlp_b1[i].reshape(1, 2 * D))
        z2, bm2, m22 = _mlp2(z1, bm1.reshape(NB, 2 * D), m21,
                             mlp_g[i].reshape(1, 2 * D),
                             mlp_beta[i].reshape(1, 2 * D), mlp_W2[i],
                             mlp_b2[i].reshape(1, D))
        h = _mlp3(z2, bm2.reshape(NB, D), m22, bn_g[i].reshape(1, D),
                  bn_beta[i].reshape(1, D), relu=(i < L - 1))

    return _pool(h, batch.reshape(1, N))


# speedup vs baseline: 4.8226x; 1.1007x over previous
import functools

import jax
import jax.numpy as jnp
from jax import lax
from jax.experimental import pallas as pl
from jax.experimental.pallas import tpu as pltpu
from jax.experimental.pallas import tpu_sc as plsc

N = 10000
E = 640000
ATOM = 101
BOND = 11
D = 128
L = 3
G = 256

NC = 2
NS = 16
NW = NC * NS
CH = 64
EPAD = 643072
TPT = EPAD // NW
NCHUNK = TPT // CH
NPAIR = NCHUNK // 2
NP = 10240
RPT = NP // NS
F32 = jnp.float32


def _embed(x, w, b):
    def body(x_ref, w_ref, b_ref, o_ref):
        o_ref[:] = (
            jnp.dot(x_ref[:], w_ref[:], preferred_element_type=F32) + b_ref[:]
        )

    R = 1000
    return pl.pallas_call(
        body,
        grid=(N // R,),
        in_specs=[
            pl.BlockSpec((R, ATOM), lambda i: (i, 0)),
            pl.BlockSpec((ATOM, D), lambda i: (0, 0)),
            pl.BlockSpec((1, D), lambda i: (0, 0)),
        ],
        out_specs=pl.BlockSpec((R, D), lambda i: (i, 0)),
        out_shape=jax.ShapeDtypeStruct((N, D), F32),
    )(x, w, b)


def _bond(attr, w, b):
    B = 4096

    def body(a_ref, w_ref, b_ref, o_ref):
        o_ref[:] = (
            jnp.dot(a_ref[:], w_ref[:], preferred_element_type=F32) + b_ref[:]
        )

    return pl.pallas_call(
        body,
        grid=(EPAD // B,),
        in_specs=[
            pl.BlockSpec((B, BOND), lambda j: (j, 0)),
            pl.BlockSpec((BOND, D), lambda j: (0, 0)),
            pl.BlockSpec((1, D), lambda j: (0, 0)),
        ],
        out_specs=pl.BlockSpec((B, D), lambda j: (j, 0)),
        out_shape=jax.ShapeDtypeStruct((EPAD, D), F32),
    )(attr, w, b)


R = 1000
NB = N // R


def _bn_stats(vals, i, bm_ref, m2_ref):
    mb = jnp.mean(vals, axis=0, keepdims=True)
    c = vals - mb
    bm_ref[0] = mb
    m2 = jnp.sum(c * c, axis=0, keepdims=True)

    @pl.when(i == 0)
    def _():
        m2_ref[:] = jnp.zeros_like(m2_ref)

    m2_ref[:] += m2


def _bn_apply(vals, bm, m2, g, beta):
    mean = jnp.mean(bm, axis=0, keepdims=True)
    dm = bm - mean
    var = m2 * (1.0 / N) + jnp.mean(dm * dm, axis=0, keepdims=True)
    return (vals - mean) * lax.rsqrt(var + 1e-5) * g + beta


def _mlp1(h, aggr, w1, b1):
    def body(h_ref, a_ref, w_ref, b_ref, z_ref, bm_ref, m2_ref):
        z = h_ref[:] + a_ref[0] + a_ref[1]
        z1 = jnp.dot(z, w_ref[:], preferred_element_type=F32) + b_ref[:]
        z_ref[:] = z1
        _bn_stats(z1, pl.program_id(0), bm_ref, m2_ref)

    return pl.pallas_call(
        body,
        grid=(NB,),
        in_specs=[
            pl.BlockSpec((R, D), lambda i: (i, 0)),
            pl.BlockSpec((2, R, D), lambda i: (0, i, 0)),
            pl.BlockSpec((D, 2 * D), lambda i: (0, 0)),
            pl.BlockSpec((1, 2 * D), lambda i: (0, 0)),
        ],
        out_specs=[
            pl.BlockSpec((R, 2 * D), lambda i: (i, 0)),
            pl.BlockSpec((1, 1, 2 * D), lambda i: (i, 0, 0)),
            pl.BlockSpec((1, 2 * D), lambda i: (0, 0)),
        ],
        out_shape=[
            jax.ShapeDtypeStruct((N, 2 * D), F32),
            jax.ShapeDtypeStruct((NB, 1, 2 * D), F32),
            jax.ShapeDtypeStruct((1, 2 * D), F32),
        ],
    )(h, aggr, w1, b1)


def _mlp2(z1, bm1, m21, g, beta, w2, b2):
    def body(z_ref, bm1_ref, m21_ref, g_ref, be_ref, w_ref, b_ref,
             o_ref, bm_ref, m2_ref):
        zn = _bn_apply(z_ref[:], bm1_ref[:], m21_ref[:], g_ref[:], be_ref[:])
        r = jnp.maximum(zn, 0.0)
        z2 = jnp.dot(r, w_ref[:], preferred_element_type=F32) + b_ref[:]
        o_ref[:] = z2
        _bn_stats(z2, pl.program_id(0), bm_ref, m2_ref)

    return pl.pallas_call(
        body,
        grid=(NB,),
        in_specs=[
            pl.BlockSpec((R, 2 * D), lambda i: (i, 0)),
            pl.BlockSpec((NB, 2 * D), lambda i: (0, 0)),
            pl.BlockSpec((1, 2 * D), lambda i: (0, 0)),
            pl.BlockSpec((1, 2 * D), lambda i: (0, 0)),
            pl.BlockSpec((1, 2 * D), lambda i: (0, 0)),
            pl.BlockSpec((2 * D, D), lambda i: (0, 0)),
            pl.BlockSpec((1, D), lambda i: (0, 0)),
        ],
        out_specs=[
            pl.BlockSpec((R, D), lambda i: (i, 0)),
            pl.BlockSpec((1, 1, D), lambda i: (i, 0, 0)),
            pl.BlockSpec((1, D), lambda i: (0, 0)),
        ],
        out_shape=[
            jax.ShapeDtypeStruct((N, D), F32),
            jax.ShapeDtypeStruct((NB, 1, D), F32),
            jax.ShapeDtypeStruct((1, D), F32),
        ],
    )(z1, bm1, m21, g, beta, w2, b2)


def _mlp3(z2, bm2, m22, g, beta, relu):
    def body(z_ref, bm_ref, m2_ref, g_ref, be_ref, o_ref):
        hn = _bn_apply(z_ref[:], bm_ref[:], m2_ref[:], g_ref[:], be_ref[:])
        if relu:
            hn = jnp.maximum(hn, 0.0)
        o_ref[:] = hn

    return pl.pallas_call(
        body,
        grid=(NB,),
        in_specs=[
            pl.BlockSpec((R, D), lambda i: (i, 0)),
            pl.BlockSpec((NB, D), lambda i: (0, 0)),
            pl.BlockSpec((1, D), lambda i: (0, 0)),
            pl.BlockSpec((1, D), lambda i: (0, 0)),
            pl.BlockSpec((1, D), lambda i: (0, 0)),
        ],
        out_specs=pl.BlockSpec((R, D), lambda i: (i, 0)),
        out_shape=jax.ShapeDtypeStruct((N, D), F32),
    )(z2, bm2, m22, g, beta)


def _pool(h, batch2d):
    GB = 64

    def body(h_ref, b_ref, o_ref):
        gbase = pl.program_id(0) * GB
        rows = lax.broadcasted_iota(jnp.int32, (GB, N), 0) + gbase
        mask = (rows == b_ref[:]).astype(F32)
        sums = jnp.dot(mask, h_ref[:], preferred_element_type=F32)
        counts = jnp.sum(mask, axis=1, keepdims=True)
        o_ref[:] = sums / jnp.maximum(counts, 1.0)

    return pl.pallas_call(
        body,
        grid=(G // GB,),
        in_specs=[
            pl.BlockSpec((N, D), lambda i: (0, 0)),
            pl.BlockSpec((1, N), lambda i: (0, 0)),
        ],
        out_specs=pl.BlockSpec((GB, D), lambda i: (i, 0)),
        out_shape=jax.ShapeDtypeStruct((G, D), F32),
    )(h, batch2d)


def _messages(h, e, src, dst):
    mesh = plsc.VectorSubcoreMesh(core_axis_name="c", subcore_axis_name="s")

    @functools.partial(
        pl.kernel,
        mesh=mesh,
        out_type=jax.ShapeDtypeStruct((NC, NP, D), F32),
        scratch_types=[
            pltpu.VMEM((CH,), jnp.int32),
            pltpu.VMEM((CH,), jnp.int32),
            pltpu.VMEM((CH,), jnp.int32),
            pltpu.VMEM((CH,), jnp.int32),
            pltpu.VMEM((CH, D), F32),
            pltpu.VMEM((CH, D), F32),
            pltpu.VMEM((CH, D), F32),
            pltpu.VMEM((CH, D), F32),
            pltpu.VMEM_SHARED((NP, D), F32),
        ] + [pltpu.SemaphoreType.DMA] * 8,
    )
    def k(h_hbm, e_hbm, src_hbm, dst_hbm, out_hbm,
          sv0, dv0, sv1, dv1, h0, e0, h1, e1, aggr_sh,
          is0, id0, ig0, ie0, is1, id1, ig1, ie1):
        cid = lax.axis_index("c")
        sid = lax.axis_index("s")
        base = (cid * NS + sid) * TPT

        def zb(r, carry):
            for j in range(D // 16):
                e0[r, pl.ds(j * 16, 16)] = jnp.zeros((16,), F32)
            return carry

        lax.fori_loop(0, CH, zb, 0)
        for t in range(RPT // CH):
            pltpu.sync_copy(e0, aggr_sh.at[pl.ds(sid * RPT + t * CH, CH)])
        plsc.subcore_barrier()

        def fire(c, sv, dv, hb, eb, s_is, s_id, s_ig, s_ie, gather_only):
            off = base + c * CH
            if not gather_only:
                pltpu.make_async_copy(
                    src_hbm.at[pl.ds(off, CH)], sv, s_is).start()
                pltpu.make_async_copy(
                    dst_hbm.at[pl.ds(off, CH)], dv, s_id).start()
                pltpu.make_async_copy(
                    e_hbm.at[pl.ds(off, CH)], eb, s_ie).start()
            else:
                pltpu.make_async_copy(
                    src_hbm.at[pl.ds(off, CH)], sv, s_is).wait()
                pltpu.make_async_copy(h_hbm.at[sv], hb, s_ig).start()

        def process(c, sv, dv, hb, eb, s_is, s_id, s_ig, s_ie):
            off = base + c * CH
            pltpu.make_async_copy(h_hbm.at[sv], hb, s_ig).wait()
            pltpu.make_async_copy(
                e_hbm.at[pl.ds(off, CH)], eb, s_ie).wait()

            def msg(r, c2):
                for j in range(D // 16):
                    s = pl.ds(j * 16, 16)
                    eb[r, s] = jnp.maximum(hb[r, s] + eb[r, s], 0.0)
                return c2

            lax.fori_loop(0, CH, msg, 0)
            pltpu.make_async_copy(
                dst_hbm.at[pl.ds(off, CH)], dv, s_id).wait()
            pltpu.sync_copy(eb, aggr_sh.at[dv], add=True)

        fire(0, sv0, dv0, h0, e0, is0, id0, ig0, ie0, False)
        fire(1, sv1, dv1, h1, e1, is1, id1, ig1, ie1, False)
        fire(0, sv0, dv0, h0, e0, is0, id0, ig0, ie0, True)
        fire(1, sv1, dv1, h1, e1, is1, id1, ig1, ie1, True)

        def pair(p, carry):
            a = 2 * p
            process(a, sv0, dv0, h0, e0, is0, id0, ig0, ie0)

            @pl.when(p < NPAIR - 1)
            def _():
                fire(a + 2, sv0, dv0, h0, e0, is0, id0, ig0, ie0, False)
                fire(a + 2, sv0, dv0, h0, e0, is0, id0, ig0, ie0, True)

            process(a + 1, sv1, dv1, h1, e1, is1, id1, ig1, ie1)

            @pl.when(p < NPAIR - 1)
            def _():
                fire(a + 3, sv1, dv1, h1, e1, is1, id1, ig1, ie1, False)
                fire(a + 3, sv1, dv1, h1, e1, is1, id1, ig1, ie1, True)

            return carry

        lax.fori_loop(0, NPAIR, pair, 0)
        plsc.subcore_barrier()
        for t in range(RPT // CH):
            r0 = sid * RPT + t * CH
            pltpu.sync_copy(aggr_sh.at[pl.ds(r0, CH)],
                            out_hbm.at[cid, pl.ds(r0, CH)])

    return k(h, e, src, dst)


def kernel(x, edge_index, edge_attr, batch, W_emb, b_emb, bond_W, bond_b,
           mlp_W1, mlp_b1, mlp_g, mlp_beta, mlp_W2, mlp_b2, bn_g, bn_beta):
    pad = EPAD - E
    src = jnp.concatenate([edge_index[0], jnp.zeros((pad,), jnp.int32)])
    dst = jnp.concatenate([edge_index[1], jnp.full((pad,), N, jnp.int32)])
    attr = jnp.concatenate(
        [edge_attr, jnp.zeros((pad, BOND), F32)], axis=0)

    h = _embed(x, W_emb, b_emb.reshape(1, D))
    e = _bond(attr, bond_W[0], bond_b[0].reshape(1, D))

    for i in range(L):
        aggr = _messages(h, e, src, dst)
        if i + 1 < L:
            e = _bond(attr, bond_W[i + 1], bond_b[i + 1].reshape(1, D))
        z1, bm1, m21 = _mlp1(h, aggr, mlp_W1[i], mlp_b1[i].reshape(1, 2 * D))
        z2, bm2, m22 = _mlp2(z1, bm1.reshape(NB, 2 * D), m21,
                             mlp_g[i].reshape(1, 2 * D),
                             mlp_beta[i].reshape(1, 2 * D), mlp_W2[i],
                             mlp_b2[i].reshape(1, D))
        h = _mlp3(z2, bm2.reshape(NB, D), m22, bn_g[i].reshape(1, D),
                  bn_beta[i].reshape(1, D), relu=(i < L - 1))

    return _pool(h, batch.reshape(1, N))


# speedup vs baseline: 4.9648x; 1.0295x over previous
import functools

import jax
import jax.numpy as jnp
from jax import lax
from jax.experimental import pallas as pl
from jax.experimental.pallas import tpu as pltpu
from jax.experimental.pallas import tpu_sc as plsc

N = 10000
E = 640000
ATOM = 101
BOND = 11
D = 128
L = 3
G = 256

NC = 2
NS = 16
NW = NC * NS
CH = 64
EPAD = 643072
TPT = EPAD // NW
NCHUNK = TPT // CH
NPAIR = NCHUNK // 2
NP = 10240
RPT = NP // NS
F32 = jnp.float32


def _embed(x, w, b):
    def body(x_ref, w_ref, b_ref, o_ref):
        o_ref[:] = (
            jnp.dot(x_ref[:], w_ref[:], preferred_element_type=F32) + b_ref[:]
        )

    R = 1000
    return pl.pallas_call(
        body,
        grid=(N // R,),
        in_specs=[
            pl.BlockSpec((R, ATOM), lambda i: (i, 0)),
            pl.BlockSpec((ATOM, D), lambda i: (0, 0)),
            pl.BlockSpec((1, D), lambda i: (0, 0)),
        ],
        out_specs=pl.BlockSpec((R, D), lambda i: (i, 0)),
        out_shape=jax.ShapeDtypeStruct((N, D), F32),
    )(x, w, b)


def _bond(attr, w, b):
    B = 4096

    def body(a_ref, w_ref, b_ref, o_ref):
        o_ref[:] = (
            jnp.dot(a_ref[:], w_ref[:], preferred_element_type=F32) + b_ref[:]
        )

    return pl.pallas_call(
        body,
        grid=(EPAD // B,),
        in_specs=[
            pl.BlockSpec((B, BOND), lambda j: (j, 0)),
            pl.BlockSpec((BOND, D), lambda j: (0, 0)),
            pl.BlockSpec((1, D), lambda j: (0, 0)),
        ],
        out_specs=pl.BlockSpec((B, D), lambda j: (j, 0)),
        out_shape=jax.ShapeDtypeStruct((EPAD, D), F32),
    )(attr, w, b)


R = 1000
NB = N // R


def _mlp_fused(h, aggr, w1, b1, g1, be1, w2, b2, g2, be2, relu_last):
    def bn(v, g, be):
        m = jnp.mean(v, axis=0, keepdims=True)
        c = v - m
        var = jnp.mean(c * c, axis=0, keepdims=True)
        return c * lax.rsqrt(var + 1e-5) * g + be

    def body(h_ref, a_ref, w1_ref, b1_ref, g1_ref, be1_ref,
             w2_ref, b2_ref, g2_ref, be2_ref, o_ref):
        z = h_ref[:] + a_ref[0, :N] + a_ref[1, :N]
        z1 = jnp.dot(z, w1_ref[:], preferred_element_type=F32) + b1_ref[:]
        z1n = jnp.maximum(bn(z1, g1_ref[:], be1_ref[:]), 0.0)
        z2 = jnp.dot(z1n, w2_ref[:], preferred_element_type=F32) + b2_ref[:]
        hn = bn(z2, g2_ref[:], be2_ref[:])
        if relu_last:
            hn = jnp.maximum(hn, 0.0)
        o_ref[:] = hn

    return pl.pallas_call(
        body,
        out_shape=jax.ShapeDtypeStruct((N, D), F32),
    )(h, aggr, w1, b1, g1, be1, w2, b2, g2, be2)


def _bn_stats(vals, i, bm_ref, m2_ref):
    mb = jnp.mean(vals, axis=0, keepdims=True)
    c = vals - mb
    bm_ref[0] = mb
    m2 = jnp.sum(c * c, axis=0, keepdims=True)

    @pl.when(i == 0)
    def _():
        m2_ref[:] = jnp.zeros_like(m2_ref)

    m2_ref[:] += m2


def _bn_apply(vals, bm, m2, g, beta):
    mean = jnp.mean(bm, axis=0, keepdims=True)
    dm = bm - mean
    var = m2 * (1.0 / N) + jnp.mean(dm * dm, axis=0, keepdims=True)
    return (vals - mean) * lax.rsqrt(var + 1e-5) * g + beta


def _mlp1(h, aggr, w1, b1):
    def body(h_ref, a_ref, w_ref, b_ref, z_ref, bm_ref, m2_ref):
        z = h_ref[:] + a_ref[0] + a_ref[1]
        z1 = jnp.dot(z, w_ref[:], preferred_element_type=F32) + b_ref[:]
        z_ref[:] = z1
        _bn_stats(z1, pl.program_id(0), bm_ref, m2_ref)

    return pl.pallas_call(
        body,
        grid=(NB,),
        in_specs=[
            pl.BlockSpec((R, D), lambda i: (i, 0)),
            pl.BlockSpec((2, R, D), lambda i: (0, i, 0)),
            pl.BlockSpec((D, 2 * D), lambda i: (0, 0)),
            pl.BlockSpec((1, 2 * D), lambda i: (0, 0)),
        ],
        out_specs=[
            pl.BlockSpec((R, 2 * D), lambda i: (i, 0)),
            pl.BlockSpec((1, 1, 2 * D), lambda i: (i, 0, 0)),
            pl.BlockSpec((1, 2 * D), lambda i: (0, 0)),
        ],
        out_shape=[
            jax.ShapeDtypeStruct((N, 2 * D), F32),
            jax.ShapeDtypeStruct((NB, 1, 2 * D), F32),
            jax.ShapeDtypeStruct((1, 2 * D), F32),
        ],
    )(h, aggr, w1, b1)


def _mlp2(z1, bm1, m21, g, beta, w2, b2):
    def body(z_ref, bm1_ref, m21_ref, g_ref, be_ref, w_ref, b_ref,
             o_ref, bm_ref, m2_ref):
        zn = _bn_apply(z_ref[:], bm1_ref[:], m21_ref[:], g_ref[:], be_ref[:])
        r = jnp.maximum(zn, 0.0)
        z2 = jnp.dot(r, w_ref[:], preferred_element_type=F32) + b_ref[:]
        o_ref[:] = z2
        _bn_stats(z2, pl.program_id(0), bm_ref, m2_ref)

    return pl.pallas_call(
        body,
        grid=(NB,),
        in_specs=[
            pl.BlockSpec((R, 2 * D), lambda i: (i, 0)),
            pl.BlockSpec((NB, 2 * D), lambda i: (0, 0)),
            pl.BlockSpec((1, 2 * D), lambda i: (0, 0)),
            pl.BlockSpec((1, 2 * D), lambda i: (0, 0)),
            pl.BlockSpec((1, 2 * D), lambda i: (0, 0)),
            pl.BlockSpec((2 * D, D), lambda i: (0, 0)),
            pl.BlockSpec((1, D), lambda i: (0, 0)),
        ],
        out_specs=[
            pl.BlockSpec((R, D), lambda i: (i, 0)),
            pl.BlockSpec((1, 1, D), lambda i: (i, 0, 0)),
            pl.BlockSpec((1, D), lambda i: (0, 0)),
        ],
        out_shape=[
            jax.ShapeDtypeStruct((N, D), F32),
            jax.ShapeDtypeStruct((NB, 1, D), F32),
            jax.ShapeDtypeStruct((1, D), F32),
        ],
    )(z1, bm1, m21, g, beta, w2, b2)


def _mlp3(z2, bm2, m22, g, beta, relu):
    def body(z_ref, bm_ref, m2_ref, g_ref, be_ref, o_ref):
        hn = _bn_apply(z_ref[:], bm_ref[:], m2_ref[:], g_ref[:], be_ref[:])
        if relu:
            hn = jnp.maximum(hn, 0.0)
        o_ref[:] = hn

    return pl.pallas_call(
        body,
        grid=(NB,),
        in_specs=[
            pl.BlockSpec((R, D), lambda i: (i, 0)),
            pl.BlockSpec((NB, D), lambda i: (0, 0)),
            pl.BlockSpec((1, D), lambda i: (0, 0)),
            pl.BlockSpec((1, D), lambda i: (0, 0)),
            pl.BlockSpec((1, D), lambda i: (0, 0)),
        ],
        out_specs=pl.BlockSpec((R, D), lambda i: (i, 0)),
        out_shape=jax.ShapeDtypeStruct((N, D), F32),
    )(z2, bm2, m22, g, beta)


def _pool(h, batch2d):
    GB = 64

    def body(h_ref, b_ref, o_ref):
        gbase = pl.program_id(0) * GB
        rows = lax.broadcasted_iota(jnp.int32, (GB, N), 0) + gbase
        mask = (rows == b_ref[:]).astype(F32)
        sums = jnp.dot(mask, h_ref[:], preferred_element_type=F32)
        counts = jnp.sum(mask, axis=1, keepdims=True)
        o_ref[:] = sums / jnp.maximum(counts, 1.0)

    return pl.pallas_call(
        body,
        grid=(G // GB,),
        in_specs=[
            pl.BlockSpec((N, D), lambda i: (0, 0)),
            pl.BlockSpec((1, N), lambda i: (0, 0)),
        ],
        out_specs=pl.BlockSpec((GB, D), lambda i: (i, 0)),
        out_shape=jax.ShapeDtypeStruct((G, D), F32),
    )(h, batch2d)


def _messages(h, e, src, dst):
    mesh = plsc.VectorSubcoreMesh(core_axis_name="c", subcore_axis_name="s")

    @functools.partial(
        pl.kernel,
        mesh=mesh,
        out_type=jax.ShapeDtypeStruct((NC, NP, D), F32),
        scratch_types=[
            pltpu.VMEM((CH,), jnp.int32),
            pltpu.VMEM((CH,), jnp.int32),
            pltpu.VMEM((CH,), jnp.int32),
            pltpu.VMEM((CH,), jnp.int32),
            pltpu.VMEM((CH, D), F32),
            pltpu.VMEM((CH, D), F32),
            pltpu.VMEM((CH, D), F32),
            pltpu.VMEM((CH, D), F32),
            pltpu.VMEM_SHARED((NP, D), F32),
        ] + [pltpu.SemaphoreType.DMA] * 8,
    )
    def k(h_hbm, e_hbm, src_hbm, dst_hbm, out_hbm,
          sv0, dv0, sv1, dv1, h0, e0, h1, e1, aggr_sh,
          is0, id0, ig0, ie0, is1, id1, ig1, ie1):
        cid = lax.axis_index("c")
        sid = lax.axis_index("s")
        base = (cid * NS + sid) * TPT

        def zb(r, carry):
            for j in range(D // 16):
                e0[r, pl.ds(j * 16, 16)] = jnp.zeros((16,), F32)
            return carry

        lax.fori_loop(0, CH, zb, 0)
        for t in range(RPT // CH):
            pltpu.sync_copy(e0, aggr_sh.at[pl.ds(sid * RPT + t * CH, CH)])
        plsc.subcore_barrier()

        def fire(c, sv, dv, hb, eb, s_is, s_id, s_ig, s_ie, gather_only):
            off = base + c * CH
            if not gather_only:
                pltpu.make_async_copy(
                    src_hbm.at[pl.ds(off, CH)], sv, s_is).start()
                pltpu.make_async_copy(
                    dst_hbm.at[pl.ds(off, CH)], dv, s_id).start()
                pltpu.make_async_copy(
                    e_hbm.at[pl.ds(off, CH)], eb, s_ie).start()
            else:
                pltpu.make_async_copy(
                    src_hbm.at[pl.ds(off, CH)], sv, s_is).wait()
                pltpu.make_async_copy(h_hbm.at[sv], hb, s_ig).start()

        def process(c, sv, dv, hb, eb, s_is, s_id, s_ig, s_ie):
            off = base + c * CH
            pltpu.make_async_copy(h_hbm.at[sv], hb, s_ig).wait()
            pltpu.make_async_copy(
                e_hbm.at[pl.ds(off, CH)], eb, s_ie).wait()

            def msg(r, c2):
                for j in range(D // 16):
                    s = pl.ds(j * 16, 16)
                    eb[r, s] = jnp.maximum(hb[r, s] + eb[r, s], 0.0)
                return c2

            lax.fori_loop(0, CH, msg, 0)
            pltpu.make_async_copy(
                dst_hbm.at[pl.ds(off, CH)], dv, s_id).wait()
            pltpu.sync_copy(eb, aggr_sh.at[dv], add=True)

        fire(0, sv0, dv0, h0, e0, is0, id0, ig0, ie0, False)
        fire(1, sv1, dv1, h1, e1, is1, id1, ig1, ie1, False)
        fire(0, sv0, dv0, h0, e0, is0, id0, ig0, ie0, True)
        fire(1, sv1, dv1, h1, e1, is1, id1, ig1, ie1, True)

        def pair(p, carry):
            a = 2 * p
            process(a, sv0, dv0, h0, e0, is0, id0, ig0, ie0)

            @pl.when(p < NPAIR - 1)
            def _():
                fire(a + 2, sv0, dv0, h0, e0, is0, id0, ig0, ie0, False)
                fire(a + 2, sv0, dv0, h0, e0, is0, id0, ig0, ie0, True)

            process(a + 1, sv1, dv1, h1, e1, is1, id1, ig1, ie1)

            @pl.when(p < NPAIR - 1)
            def _():
                fire(a + 3, sv1, dv1, h1, e1, is1, id1, ig1, ie1, False)
                fire(a + 3, sv1, dv1, h1, e1, is1, id1, ig1, ie1, True)

            return carry

        lax.fori_loop(0, NPAIR, pair, 0)
        plsc.subcore_barrier()
        for t in range(RPT // CH):
            r0 = sid * RPT + t * CH
            pltpu.sync_copy(aggr_sh.at[pl.ds(r0, CH)],
                            out_hbm.at[cid, pl.ds(r0, CH)])

    return k(h, e, src, dst)


def kernel(x, edge_index, edge_attr, batch, W_emb, b_emb, bond_W, bond_b,
           mlp_W1, mlp_b1, mlp_g, mlp_beta, mlp_W2, mlp_b2, bn_g, bn_beta):
    pad = EPAD - E
    src = jnp.concatenate([edge_index[0], jnp.zeros((pad,), jnp.int32)])
    dst = jnp.concatenate([edge_index[1], jnp.full((pad,), N, jnp.int32)])
    attr = jnp.concatenate(
        [edge_attr, jnp.zeros((pad, BOND), F32)], axis=0)

    h = _embed(x, W_emb, b_emb.reshape(1, D))
    e = _bond(attr, bond_W[0], bond_b[0].reshape(1, D))

    for i in range(L):
        aggr = _messages(h, e, src, dst)
        if i + 1 < L:
            e = _bond(attr, bond_W[i + 1], bond_b[i + 1].reshape(1, D))
        h = _mlp_fused(h, aggr, mlp_W1[i], mlp_b1[i].reshape(1, 2 * D),
                       mlp_g[i].reshape(1, 2 * D),
                       mlp_beta[i].reshape(1, 2 * D), mlp_W2[i],
                       mlp_b2[i].reshape(1, D), bn_g[i].reshape(1, D),
                       bn_beta[i].reshape(1, D), relu_last=(i < L - 1))

    return _pool(h, batch.reshape(1, N))
